# Initial kernel scaffold; baseline (speedup 1.0000x reference)
import jax
import jax.numpy as jnp
from jax.experimental import pallas as pl


def kernel(boxes, scores):
    raise NotImplementedError("write your pallas kernel here")


# speedup vs baseline: 37.2768x; 37.2768x over previous
import jax
import jax.numpy as jnp
from jax.experimental import pallas as pl
from jax.experimental.pallas import tpu as pltpu

_N = 4096
_T = 256
_B = _N // _T
_THR = 0.5


def _nms_body(boxes_ref, boxes_t_ref, probs_ref, out_ref,
              keep_row_ref, keep_col_ref, diag_ref):
    riota = jax.lax.broadcasted_iota(jnp.int32, (_T, _T), 0)
    ciota = jax.lax.broadcasted_iota(jnp.int32, (_T, _T), 1)
    eye = riota == ciota
    col_iota1 = jax.lax.broadcasted_iota(jnp.int32, (1, _T), 1)

    def iou_tile(base_r, base_c):
        x1c = boxes_ref[pl.ds(base_r, _T), 0:1]
        y1c = boxes_ref[pl.ds(base_r, _T), 1:2]
        x2c = boxes_ref[pl.ds(base_r, _T), 2:3]
        y2c = boxes_ref[pl.ds(base_r, _T), 3:4]
        x1r = boxes_t_ref[0:1, pl.ds(base_c, _T)]
        y1r = boxes_t_ref[1:2, pl.ds(base_c, _T)]
        x2r = boxes_t_ref[2:3, pl.ds(base_c, _T)]
        y2r = boxes_t_ref[3:4, pl.ds(base_c, _T)]
        area_c = (x2c - x1c) * (y2c - y1c)
        area_r = (x2r - x1r) * (y2r - y1r)
        w = jnp.maximum(jnp.minimum(x2c, x2r) - jnp.maximum(x1c, x1r), 0.0)
        h = jnp.maximum(jnp.minimum(y2c, y2r) - jnp.maximum(y1c, y1r), 0.0)
        inter = w * h
        union = area_c + area_r - inter
        return inter / jnp.maximum(union, 1e-9)

    def block_step(bj, carry):
        base_j = bj * _T

        def cross(bi, sup):
            base_i = bi * _T
            iou = iou_tile(base_i, base_j)
            ki = keep_col_ref[pl.ds(base_i, _T), 0:1]
            hit = jnp.where((ki > 0.0) & (iou > _THR), 1.0, 0.0)
            return jnp.maximum(sup, jnp.max(hit, axis=0, keepdims=True))

        sup = jax.lax.fori_loop(0, bj, cross, jnp.zeros((1, _T), jnp.float32))
        keep_j = 1.0 - sup

        diag_ref[...] = iou_tile(base_j, base_j)

        def inner(j, kp):
            srow = diag_ref[pl.ds(j, 1), :]
            kj = jnp.max(jnp.where(col_iota1 == j, kp, 0.0))
            sup_j = (srow > _THR) & (col_iota1 > j) & (kj > 0.0)
            return jnp.where(sup_j, 0.0, kp)

        keep_j = jax.lax.fori_loop(0, _T, inner, keep_j)

        keep_row_ref[0:1, pl.ds(base_j, _T)] = keep_j
        kcol = jnp.max(jnp.where(eye, keep_j, 0.0), axis=1, keepdims=True)
        keep_col_ref[pl.ds(base_j, _T), 0:1] = kcol
        return carry

    jax.lax.fori_loop(0, _B, block_step, 0)

    kr = keep_row_ref[0:1, :]
    out_ref[0:1, :] = boxes_t_ref[0:1, :] * kr
    out_ref[1:2, :] = boxes_t_ref[1:2, :] * kr
    out_ref[2:3, :] = boxes_t_ref[2:3, :] * kr
    out_ref[3:4, :] = boxes_t_ref[3:4, :] * kr
    out_ref[4:5, :] = probs_ref[0:1, :] * kr


def _nms_call(top_boxes, boxes_t, probs_row, interpret=False):
    return pl.pallas_call(
        _nms_body,
        out_shape=jax.ShapeDtypeStruct((5, _N), jnp.float32),
        scratch_shapes=[
            pltpu.VMEM((1, _N), jnp.float32),
            pltpu.VMEM((_N, 1), jnp.float32),
            pltpu.VMEM((_T, _T), jnp.float32),
        ],
        interpret=interpret,
    )(top_boxes, boxes_t, probs_row)


def kernel(boxes, scores):
    probs = jax.nn.sigmoid(scores)
    top_probs, top_idx = jax.lax.top_k(probs, _N)
    top_boxes = boxes[top_idx]
    out_t = _nms_call(top_boxes, top_boxes.T, top_probs[None, :])
    return out_t.T


# speedup vs baseline: 205.4997x; 5.5128x over previous
import jax
import jax.numpy as jnp
from jax.experimental import pallas as pl
from jax.experimental.pallas import tpu as pltpu

_N = 4096
_T = 256
_B = _N // _T
_THR = 0.5


def _nms_body(boxes_ref, boxes_t_ref, probs_ref, out_ref,
              keep_row_ref, keep_col_ref):
    riota = jax.lax.broadcasted_iota(jnp.int32, (_T, _T), 0)
    ciota = jax.lax.broadcasted_iota(jnp.int32, (_T, _T), 1)
    eye = riota == ciota
    tri = riota < ciota

    def iou_tile(base_r, base_c):
        x1c = boxes_ref[pl.ds(base_r, _T), 0:1]
        y1c = boxes_ref[pl.ds(base_r, _T), 1:2]
        x2c = boxes_ref[pl.ds(base_r, _T), 2:3]
        y2c = boxes_ref[pl.ds(base_r, _T), 3:4]
        x1r = boxes_t_ref[0:1, pl.ds(base_c, _T)]
        y1r = boxes_t_ref[1:2, pl.ds(base_c, _T)]
        x2r = boxes_t_ref[2:3, pl.ds(base_c, _T)]
        y2r = boxes_t_ref[3:4, pl.ds(base_c, _T)]
        area_c = (x2c - x1c) * (y2c - y1c)
        area_r = (x2r - x1r) * (y2r - y1r)
        w = jnp.maximum(jnp.minimum(x2c, x2r) - jnp.maximum(x1c, x1r), 0.0)
        h = jnp.maximum(jnp.minimum(y2c, y2r) - jnp.maximum(y1c, y1r), 0.0)
        inter = w * h
        union = area_c + area_r - inter
        return inter / jnp.maximum(union, 1e-9)

    def block_step(bj, carry):
        base_j = bj * _T

        def cross(bi, sup):
            base_i = bi * _T
            iou = iou_tile(base_i, base_j)
            ki = keep_col_ref[pl.ds(base_i, _T), 0:1]
            hit = jnp.where((ki > 0.0) & (iou > _THR), 1.0, 0.0)
            return jnp.maximum(sup, jnp.max(hit, axis=0, keepdims=True))

        sup = jax.lax.fori_loop(0, bj, cross, jnp.zeros((1, _T), jnp.float32))
        init = 1.0 - sup

        diag_hit = (iou_tile(base_j, base_j) > _THR) & tri

        def fix_body(carry):
            kp, _ = carry
            kcol = jnp.max(jnp.where(eye, kp, 0.0), axis=1, keepdims=True)
            supd = jnp.max(
                jnp.where((kcol > 0.0) & diag_hit, 1.0, 0.0),
                axis=0, keepdims=True)
            new = jnp.where(supd > 0.0, 0.0, init)
            return new, jnp.any(new != kp)

        keep_j, _ = jax.lax.while_loop(
            lambda c: c[1], fix_body, (init, jnp.bool_(True)))

        keep_row_ref[0:1, pl.ds(base_j, _T)] = keep_j
        kcol = jnp.max(jnp.where(eye, keep_j, 0.0), axis=1, keepdims=True)
        keep_col_ref[pl.ds(base_j, _T), 0:1] = kcol
        return carry

    jax.lax.fori_loop(0, _B, block_step, 0)

    kr = keep_row_ref[0:1, :]
    out_ref[0:1, :] = boxes_t_ref[0:1, :] * kr
    out_ref[1:2, :] = boxes_t_ref[1:2, :] * kr
    out_ref[2:3, :] = boxes_t_ref[2:3, :] * kr
    out_ref[3:4, :] = boxes_t_ref[3:4, :] * kr
    out_ref[4:5, :] = probs_ref[0:1, :] * kr


def _nms_call(top_boxes, boxes_t, probs_row, interpret=False):
    return pl.pallas_call(
        _nms_body,
        out_shape=jax.ShapeDtypeStruct((5, _N), jnp.float32),
        scratch_shapes=[
            pltpu.VMEM((1, _N), jnp.float32),
            pltpu.VMEM((_N, 1), jnp.float32),
        ],
        interpret=interpret,
    )(top_boxes, boxes_t, probs_row)


def kernel(boxes, scores):
    probs = jax.nn.sigmoid(scores)
    top_probs, top_idx = jax.lax.top_k(probs, _N)
    top_boxes = boxes[top_idx]
    out_t = _nms_call(top_boxes, top_boxes.T, top_probs[None, :])
    return out_t.T


# speedup vs baseline: 265.3105x; 1.2911x over previous
import jax
import jax.numpy as jnp
from jax.experimental import pallas as pl
from jax.experimental.pallas import tpu as pltpu

_N = 4096
_T = 256
_B = _N // _T
_THR = 0.5
_FAR = -1e9


def _nms_body(boxes_ref, boxes_t_ref, probs_ref, out_ref, keep_row_ref):
    riota = jax.lax.broadcasted_iota(jnp.int32, (_T, _T), 0)
    ciota = jax.lax.broadcasted_iota(jnp.int32, (_T, _T), 1)
    eye = riota == ciota
    tri = riota < ciota

    keep_row_ref[...] = jnp.ones((1, _N), jnp.float32)

    def col_comps(base):
        x1c = boxes_ref[pl.ds(base, _T), 0:1]
        y1c = boxes_ref[pl.ds(base, _T), 1:2]
        x2c = boxes_ref[pl.ds(base, _T), 2:3]
        y2c = boxes_ref[pl.ds(base, _T), 3:4]
        return x1c, y1c, x2c, y2c

    def row_comps(base):
        x1r = boxes_t_ref[0:1, pl.ds(base, _T)]
        y1r = boxes_t_ref[1:2, pl.ds(base, _T)]
        x2r = boxes_t_ref[2:3, pl.ds(base, _T)]
        y2r = boxes_t_ref[3:4, pl.ds(base, _T)]
        return x1r, y1r, x2r, y2r

    def block_step(bi, carry):
        base_i = bi * _T
        x1c, y1c, x2c, y2c = col_comps(base_i)
        x1r, y1r, x2r, y2r = row_comps(base_i)

        area_c = (x2c - x1c) * (y2c - y1c)
        area_r = (x2r - x1r) * (y2r - y1r)
        w = jnp.maximum(jnp.minimum(x2c, x2r) - jnp.maximum(x1c, x1r), 0.0)
        h = jnp.maximum(jnp.minimum(y2c, y2r) - jnp.maximum(y1c, y1r), 0.0)
        inter = w * h
        union = area_c + area_r - inter
        diag_hit = (inter / jnp.maximum(union, 1e-9) > _THR) & tri

        init = keep_row_ref[0:1, pl.ds(base_i, _T)]

        def fix_body(c):
            kp, _ = c
            kcol = jnp.max(jnp.where(eye, kp, 0.0), axis=1, keepdims=True)
            supd = jnp.max(
                jnp.where((kcol > 0.0) & diag_hit, 1.0, 0.0),
                axis=0, keepdims=True)
            new = jnp.where(supd > 0.0, 0.0, init)
            return new, jnp.any(new != kp)

        keep_i, _ = jax.lax.while_loop(
            lambda c: c[1], fix_body, (init, jnp.bool_(True)))
        keep_row_ref[0:1, pl.ds(base_i, _T)] = keep_i

        kmask = jnp.max(jnp.where(eye, keep_i, 0.0), axis=1, keepdims=True) > 0.0
        mx1 = jnp.where(kmask, x1c, _FAR)
        my1 = jnp.where(kmask, y1c, _FAR)
        mx2 = jnp.where(kmask, x2c, _FAR)
        my2 = jnp.where(kmask, y2c, _FAR)
        mac = (mx2 - mx1) * (my2 - my1)
        X1 = jnp.broadcast_to(mx1, (_T, _T))
        Y1 = jnp.broadcast_to(my1, (_T, _T))
        X2 = jnp.broadcast_to(mx2, (_T, _T))
        Y2 = jnp.broadcast_to(my2, (_T, _T))
        AC = jnp.broadcast_to(mac, (_T, _T))

        def sweep(bj, c):
            base_j = bj * _T
            x1, y1, x2, y2 = row_comps(base_j)
            ar = (x2 - x1) * (y2 - y1)
            ww = jnp.maximum(jnp.minimum(X2, x2) - jnp.maximum(X1, x1), 0.0)
            hh = jnp.maximum(jnp.minimum(Y2, y2) - jnp.maximum(Y1, y1), 0.0)
            it = ww * hh
            un = AC + ar - it
            iou = it / jnp.maximum(un, 1e-9)
            hit = jnp.max(iou, axis=0, keepdims=True)
            kb = keep_row_ref[0:1, pl.ds(base_j, _T)]
            keep_row_ref[0:1, pl.ds(base_j, _T)] = jnp.where(
                hit > _THR, 0.0, kb)
            return c

        jax.lax.fori_loop(bi + 1, _B, sweep, 0)
        return carry

    jax.lax.fori_loop(0, _B, block_step, 0)

    kr = keep_row_ref[0:1, :]
    out_ref[0:1, :] = boxes_t_ref[0:1, :] * kr
    out_ref[1:2, :] = boxes_t_ref[1:2, :] * kr
    out_ref[2:3, :] = boxes_t_ref[2:3, :] * kr
    out_ref[3:4, :] = boxes_t_ref[3:4, :] * kr
    out_ref[4:5, :] = probs_ref[0:1, :] * kr


def _nms_call(top_boxes, boxes_t, probs_row, interpret=False):
    return pl.pallas_call(
        _nms_body,
        out_shape=jax.ShapeDtypeStruct((5, _N), jnp.float32),
        scratch_shapes=[
            pltpu.VMEM((1, _N), jnp.float32),
        ],
        interpret=interpret,
    )(top_boxes, boxes_t, probs_row)


def kernel(boxes, scores):
    probs = jax.nn.sigmoid(scores)
    top_probs, top_idx = jax.lax.top_k(probs, _N)
    top_boxes = boxes[top_idx]
    out_t = _nms_call(top_boxes, top_boxes.T, top_probs[None, :])
    return out_t.T


# speedup vs baseline: 287.1961x; 1.0825x over previous
import jax
import jax.numpy as jnp
from jax.experimental import pallas as pl
from jax.experimental.pallas import tpu as pltpu

_N = 4096
_T = 512
_B = _N // _T
_THR = 0.5
_FAR = -1e9


def _nms_body(boxes_ref, boxes_t_ref, probs_ref, out_ref, keep_row_ref):
    riota = jax.lax.broadcasted_iota(jnp.int32, (_T, _T), 0)
    ciota = jax.lax.broadcasted_iota(jnp.int32, (_T, _T), 1)
    eye = riota == ciota
    tri = riota < ciota

    keep_row_ref[...] = jnp.ones((1, _N), jnp.float32)

    def col_comps(base):
        x1c = boxes_ref[pl.ds(base, _T), 0:1]
        y1c = boxes_ref[pl.ds(base, _T), 1:2]
        x2c = boxes_ref[pl.ds(base, _T), 2:3]
        y2c = boxes_ref[pl.ds(base, _T), 3:4]
        return x1c, y1c, x2c, y2c

    def row_comps(base):
        x1r = boxes_t_ref[0:1, pl.ds(base, _T)]
        y1r = boxes_t_ref[1:2, pl.ds(base, _T)]
        x2r = boxes_t_ref[2:3, pl.ds(base, _T)]
        y2r = boxes_t_ref[3:4, pl.ds(base, _T)]
        return x1r, y1r, x2r, y2r

    def block_step(bi, carry):
        base_i = bi * _T
        x1c, y1c, x2c, y2c = col_comps(base_i)
        x1r, y1r, x2r, y2r = row_comps(base_i)

        area_c = (x2c - x1c) * (y2c - y1c)
        area_r = (x2r - x1r) * (y2r - y1r)
        w = jnp.maximum(jnp.minimum(x2c, x2r) - jnp.maximum(x1c, x1r), 0.0)
        h = jnp.maximum(jnp.minimum(y2c, y2r) - jnp.maximum(y1c, y1r), 0.0)
        inter = w * h
        union = area_c + area_r - inter
        diag_hit = (inter / jnp.maximum(union, 1e-9) > _THR) & tri

        init = keep_row_ref[0:1, pl.ds(base_i, _T)]

        def fix_body(c):
            kp, _ = c
            kcol = jnp.max(jnp.where(eye, kp, 0.0), axis=1, keepdims=True)
            supd = jnp.max(
                jnp.where((kcol > 0.0) & diag_hit, 1.0, 0.0),
                axis=0, keepdims=True)
            new = jnp.where(supd > 0.0, 0.0, init)
            return new, jnp.any(new != kp)

        keep_i, _ = jax.lax.while_loop(
            lambda c: c[1], fix_body, (init, jnp.bool_(True)))
        keep_row_ref[0:1, pl.ds(base_i, _T)] = keep_i

        kmask = jnp.max(jnp.where(eye, keep_i, 0.0), axis=1, keepdims=True) > 0.0
        mx1 = jnp.where(kmask, x1c, _FAR)
        my1 = jnp.where(kmask, y1c, _FAR)
        mx2 = jnp.where(kmask, x2c, _FAR)
        my2 = jnp.where(kmask, y2c, _FAR)
        mac = (mx2 - mx1) * (my2 - my1)
        X1 = jnp.broadcast_to(mx1, (_T, _T))
        Y1 = jnp.broadcast_to(my1, (_T, _T))
        X2 = jnp.broadcast_to(mx2, (_T, _T))
        Y2 = jnp.broadcast_to(my2, (_T, _T))
        AC = jnp.broadcast_to(mac, (_T, _T))

        def sweep(bj, c):
            base_j = bj * _T
            x1, y1, x2, y2 = row_comps(base_j)
            ar = (x2 - x1) * (y2 - y1)
            ww = jnp.maximum(jnp.minimum(X2, x2) - jnp.maximum(X1, x1), 0.0)
            hh = jnp.maximum(jnp.minimum(Y2, y2) - jnp.maximum(Y1, y1), 0.0)
            it = ww * hh
            un = AC + ar - it
            iou = it / jnp.maximum(un, 1e-9)
            hit = jnp.max(iou, axis=0, keepdims=True)
            kb = keep_row_ref[0:1, pl.ds(base_j, _T)]
            keep_row_ref[0:1, pl.ds(base_j, _T)] = jnp.where(
                hit > _THR, 0.0, kb)
            return c

        jax.lax.fori_loop(bi + 1, _B, sweep, 0)
        return carry

    jax.lax.fori_loop(0, _B, block_step, 0)

    kr = keep_row_ref[0:1, :]
    out_ref[0:1, :] = boxes_t_ref[0:1, :] * kr
    out_ref[1:2, :] = boxes_t_ref[1:2, :] * kr
    out_ref[2:3, :] = boxes_t_ref[2:3, :] * kr
    out_ref[3:4, :] = boxes_t_ref[3:4, :] * kr
    out_ref[4:5, :] = probs_ref[0:1, :] * kr


def _nms_call(top_boxes, boxes_t, probs_row, interpret=False):
    return pl.pallas_call(
        _nms_body,
        out_shape=jax.ShapeDtypeStruct((5, _N), jnp.float32),
        scratch_shapes=[
            pltpu.VMEM((1, _N), jnp.float32),
        ],
        interpret=interpret,
    )(top_boxes, boxes_t, probs_row)


def kernel(boxes, scores):
    probs = jax.nn.sigmoid(scores)
    top_probs, top_idx = jax.lax.top_k(probs, _N)
    top_boxes = boxes[top_idx]
    out_t = _nms_call(top_boxes, top_boxes.T, top_probs[None, :])
    return out_t.T
